# baseline (speedup 1.0000x reference)
import functools

import jax
import jax.numpy as jnp
from jax import lax
from jax.experimental import pallas as pl
from jax.experimental.pallas import tpu as pltpu
from jax.experimental.pallas import tpu_sc as plsc

B = 1024
J = 16
NJ = B * J
NS = 3 * B
NM = 2 * B
NR = B
DJ = 128
DO = 64
H = 4
GD = 128
A = 4096


def _mm_body(x_ref, w_ref, b_ref, o_ref, *, act):
    y = jnp.dot(x_ref[...], w_ref[...], preferred_element_type=jnp.float32)
    y = y + b_ref[...]
    if act:
        y = jnp.maximum(y, 0.0)
    o_ref[...] = y


def _mm(x, W, b, act, bm=2048):
    M, K = x.shape
    N = W.shape[1]
    bvec = jnp.zeros((1, N), jnp.float32) if b is None else b.reshape(1, N)
    bm = min(bm, M)
    return pl.pallas_call(
        functools.partial(_mm_body, act=act),
        grid=(M // bm,),
        in_specs=[pl.BlockSpec((bm, K), lambda i: (i, 0)),
                  pl.BlockSpec((K, N), lambda i: (0, 0)),
                  pl.BlockSpec((1, N), lambda i: (0, 0))],
        out_specs=pl.BlockSpec((bm, N), lambda i: (i, 0)),
        out_shape=jax.ShapeDtypeStruct((M, N), jnp.float32),
    )(x, W, bvec)


def _sc_gat_multi(rels, od):
    od2 = od // 2
    TW = od2 + 16
    C = od // H
    KCH = 128
    NB = od2 // 16
    R = len(rels)
    nd_max = max(r[4] for r in rels)

    mesh = plsc.VectorSubcoreMesh(core_axis_name="c", subcore_axis_name="s")

    @functools.partial(
        pl.kernel, mesh=mesh,
        compiler_params=pltpu.CompilerParams(use_tc_tiling_on_sc=False),
        out_type=tuple(jax.ShapeDtypeStruct((2 * r[4], TW), jnp.float32)
                       for r in rels),
        scratch_types=[
            pltpu.VMEM((3, KCH), jnp.int32),
            pltpu.VMEM((3, KCH), jnp.int32),
            pltpu.VMEM((3, KCH, TW), jnp.float32),
            pltpu.VMEM((3, KCH, 16), jnp.float32),
            pltpu.VMEM((32, TW), jnp.float32),
            pltpu.VMEM_SHARED((nd_max, TW), jnp.float32),
        ] + [pltpu.SemaphoreType.DMA] * 6,
    )
    def k(*refs):
        ins = refs[:4 * R]
        outs = refs[4 * R:5 * R]
        isrc, idst, rows, adrows, zbuf, acc = refs[5 * R:5 * R + 6]
        semg = refs[5 * R + 6:5 * R + 9]
        semw = refs[5 * R + 9:5 * R + 12]
        c = lax.axis_index("c")
        s = lax.axis_index("s")

        def zrow(i, _):
            for bb in range(TW // 16):
                zbuf[i, pl.ds(bb * 16, 16)] = jnp.zeros((16,), jnp.float32)
            return 0
        lax.fori_loop(0, 32, zrow, 0)

        for r in range(R):
            src_h, dst_h, tab_h, adt_h = ins[4 * r:4 * r + 4]
            out_h = outs[r]
            E = src_h.shape[0]
            ns = tab_h.shape[0] // 2
            nd = rels[r][4]
            EPT = E // 16
            NCH = EPT // KCH
            RPT = nd // 16
            off = c * ns

            def zcp(i, _):
                pltpu.sync_copy(zbuf, acc.at[pl.ds(s * RPT + i * 32, 32)])
                return 0
            lax.fori_loop(0, RPT // 32, zcp, 0)
            plsc.subcore_barrier()

            def launch(i, p):
                base = s * EPT + i * KCH
                pltpu.sync_copy(src_h.at[pl.ds(base, KCH)], isrc.at[p])
                pltpu.sync_copy(dst_h.at[pl.ds(base, KCH)], idst.at[p])

                @plsc.parallel_loop(0, KCH // 16, 1, unroll=4)
                def addoff(j):
                    isrc[p, pl.ds(j * 16, 16)] = isrc[p, pl.ds(j * 16, 16)] + off
                pltpu.async_copy(tab_h.at[isrc.at[p]], rows.at[p], semg[p])
                pltpu.async_copy(adt_h.at[idst.at[p]], adrows.at[p], semg[p])

            def wait_scatter(p):
                pltpu.make_async_copy(rows.at[p], acc.at[idst.at[p]], semw[p]).wait()

            def process(i, p):
                pltpu.make_async_copy(tab_h.at[isrc.at[p]], rows.at[p], semg[p]).wait()
                pltpu.make_async_copy(adt_h.at[idst.at[p]], adrows.at[p], semg[p]).wait()

                @plsc.parallel_loop(0, KCH, 1, unroll=2)
                def edge(kk):
                    a_s = rows[p, kk, pl.ds(od2, 16)]
                    a_d = adrows[p, kk, :]
                    e = a_s + a_d
                    e = jnp.maximum(e, 0.2 * e)
                    w = jnp.exp(e)
                    rows[p, kk, pl.ds(od2, 16)] = w
                    w0 = jnp.where(c == 0, w[0], w[2])
                    w1 = jnp.where(c == 0, w[1], w[3])
                    for bb in range(NB):
                        ws = w0 if (bb * 16) // C == 0 else w1
                        rows[p, kk, pl.ds(bb * 16, 16)] = (
                            rows[p, kk, pl.ds(bb * 16, 16)] * ws)
                pltpu.async_copy(rows.at[p], acc.at[idst.at[p]], semw[p], add=True)

            def step(j, q):
                nb = (q + 1) % 3

                @pl.when(j >= 2)
                def _():
                    wait_scatter(nb)

                @pl.when(j + 1 < NCH)
                def _():
                    launch(j + 1, nb)
                process(j, q)

            launch(0, 0)

            def trio(i, _):
                for q in range(3):
                    step(3 * i + q, q)
                return 0
            lax.fori_loop(0, NCH // 3, trio, 0)
            for j in range(3 * (NCH // 3), NCH):
                step(jnp.int32(j), j % 3)
            for t in (NCH - 2, NCH - 1):
                wait_scatter(t % 3)
            plsc.subcore_barrier()
            pltpu.sync_copy(acc.at[pl.ds(s * RPT, RPT)],
                            out_h.at[pl.ds(c * nd + s * RPT, RPT)])

    flat = []
    for src, dst, tab, adt, _nd in rels:
        flat += [src, dst, tab, adt]
    return k(*flat)


def _sc_gather(tab, idx2d):
    D = tab.shape[1]
    mesh = plsc.VectorSubcoreMesh(core_axis_name="c", subcore_axis_name="s")

    @functools.partial(
        pl.kernel, mesh=mesh,
        compiler_params=pltpu.CompilerParams(use_tc_tiling_on_sc=False),
        out_type=jax.ShapeDtypeStruct((64 * 128, D), jnp.float32),
        scratch_types=[
            pltpu.VMEM((128,), jnp.int32),
            pltpu.VMEM((128, D), jnp.float32),
            pltpu.SemaphoreType.DMA,
        ],
    )
    def k(tab_h, idx_h, out_h, idxv, rows, sem):
        c = lax.axis_index("c")
        s = lax.axis_index("s")
        wid = s * 2 + c
        for j in range(2):
            pltpu.sync_copy(idx_h.at[wid * 2 + j], idxv)
            pltpu.async_copy(tab_h.at[idxv], rows, sem).wait()
            pltpu.sync_copy(rows, out_h.at[pl.ds((wid * 2 + j) * 128, 128)])

    return k(tab, idx2d)


def _combine_body(*refs, n_rel, od, bm):
    od2 = od // 2
    C = od // H
    part_refs = refs[:2 * n_rel]
    res_ref, bsum_ref, g_ref, b_ref, o_ref = refs[2 * n_rel:]
    cols = []
    for c in range(2):
        acc = jnp.zeros((bm, od2), jnp.float32)
        for r in range(n_rel):
            t = part_refs[2 * r + c][...]
            num = t[:, :od2]
            den = t[:, od2: od2 + 2]
            dene = jnp.concatenate(
                [jnp.broadcast_to(den[:, kk:kk + 1], (bm, C)) for kk in range(2)],
                axis=1)
            acc = acc + num / (dene + 1e-16)
        cols.append(acc)
    msg = jnp.concatenate(cols, axis=1) + bsum_ref[...]
    x = jnp.maximum(msg + res_ref[...], 0.0)
    mu = jnp.mean(x, axis=1, keepdims=True)
    xc = x - mu
    v = jnp.mean(xc * xc, axis=1, keepdims=True)
    o_ref[...] = xc / jnp.sqrt(v + 1e-5) * g_ref[...] + b_ref[...]


def _combine(parts, res, bsum, g, b, od, bm=1024):
    nd = res.shape[0]
    od2 = od // 2
    TW = od2 + 16
    n_rel = len(parts) // 2
    bm = min(bm, nd)
    pspec = [pl.BlockSpec((bm, TW), lambda i: (i, 0)) for _ in parts]
    return pl.pallas_call(
        functools.partial(_combine_body, n_rel=n_rel, od=od, bm=bm),
        grid=(nd // bm,),
        in_specs=pspec + [pl.BlockSpec((bm, od), lambda i: (i, 0)),
                          pl.BlockSpec((1, od), lambda i: (0, 0)),
                          pl.BlockSpec((1, od), lambda i: (0, 0)),
                          pl.BlockSpec((1, od), lambda i: (0, 0))],
        out_specs=pl.BlockSpec((bm, od), lambda i: (i, 0)),
        out_shape=jax.ShapeDtypeStruct((nd, od), jnp.float32),
    )(*parts, res, bsum.reshape(1, od), g.reshape(1, od), b.reshape(1, od))


def _pool_body(hjg_ref, hn_ref, wg_ref, glw_ref, glb_ref, o_ref, *, gb):
    hjg = hjg_ref[...]
    gate = jnp.dot(hjg, wg_ref[...], preferred_element_type=jnp.float32)
    gate = gate - jnp.max(gate, axis=1, keepdims=True)
    ge = jnp.exp(gate)
    w = ge / (jnp.sum(ge, axis=1, keepdims=True) + 1e-16)
    mj = jnp.zeros((gb, DJ), jnp.float32)
    for j in range(J):
        mj = mj + w[:, j:j + 1] * hjg[:, j * DJ:(j + 1) * DJ]
    feat = jnp.concatenate([hn_ref[...], mj], axis=1)
    y = jnp.dot(feat, glw_ref[...], preferred_element_type=jnp.float32)
    o_ref[...] = jnp.maximum(y + glb_ref[...], 0.0)


def _pool_global(hjg, hn, wg_big, gl_W, gl_b, gb=256):
    return pl.pallas_call(
        functools.partial(_pool_body, gb=gb),
        grid=(B // gb,),
        in_specs=[pl.BlockSpec((gb, J * DJ), lambda i: (i, 0)),
                  pl.BlockSpec((gb, 6 * DO), lambda i: (i, 0)),
                  pl.BlockSpec((J * DJ, J), lambda i: (0, 0)),
                  pl.BlockSpec((6 * DO + DJ, GD), lambda i: (0, 0)),
                  pl.BlockSpec((1, GD), lambda i: (0, 0))],
        out_specs=pl.BlockSpec((gb, GD), lambda i: (i, 0)),
        out_shape=jax.ShapeDtypeStruct((B, GD), jnp.float32),
    )(hjg, hn, wg_big, gl_W, gl_b.reshape(1, GD))


def _head_body(f_ref, w1, b1, w2, b2, w3, b3, o_ref):
    h = jnp.dot(f_ref[...], w1[...], preferred_element_type=jnp.float32) + b1[...]
    h = jnp.maximum(h, 0.0)
    h = jnp.dot(h, w2[...], preferred_element_type=jnp.float32) + b2[...]
    h = jnp.maximum(h, 0.0)
    o_ref[...] = jnp.dot(h, w3[...], preferred_element_type=jnp.float32) + b3[...]


def _head(feat, P, bm=2048):
    K = feat.shape[1]
    return pl.pallas_call(
        _head_body,
        grid=(A // bm,),
        in_specs=[pl.BlockSpec((bm, K), lambda i: (i, 0)),
                  pl.BlockSpec((K, 64), lambda i: (0, 0)),
                  pl.BlockSpec((1, 64), lambda i: (0, 0)),
                  pl.BlockSpec((64, 32), lambda i: (0, 0)),
                  pl.BlockSpec((1, 32), lambda i: (0, 0)),
                  pl.BlockSpec((32, 1), lambda i: (0, 0)),
                  pl.BlockSpec((1, 1), lambda i: (0, 0))],
        out_specs=pl.BlockSpec((bm, 1), lambda i: (i, 0)),
        out_shape=jax.ShapeDtypeStruct((A, 1), jnp.float32),
    )(feat, P["q1_W"], P["q1_b"].reshape(1, 64), P["q2_W"],
      P["q2_b"].reshape(1, 32), P["q3_W"], P["q3_b"].reshape(1, 1))


def _fold_attn(W, avec):
    sd, od = W.shape
    C = od // H
    return (W.reshape(sd, H, C) * avec[None]).sum(-1)


def _tabW(p):
    Ws = p["Ws"]
    od = p["b"].shape[0]
    od2 = od // 2
    Wa = jnp.pad(_fold_attn(Ws, p["as"]), ((0, 0), (0, 12)))
    return jnp.concatenate([Ws[:, :od2], Wa, Ws[:, od2:], Wa], axis=1)


def _adW(p):
    return jnp.pad(_fold_attn(p["Wd"], p["ad"]), ((0, 0), (0, 12)))


def kernel(x_job, x_station, x_machine, x_robot, alpha, actions, params, edges):
    P = params
    hj = _mm(x_job, P["lj_W"], P["lj_b"], True)
    hs = _mm(x_station, P["ls_W"], P["ls_b"], True)
    hm = _mm(x_machine, P["lm_W"], P["lm_b"], True)
    hr = _mm(x_robot, P["lr_W"], P["lr_b"], True)

    for jl, ol in (("j1", "o1"), ("j2", "o2")):
        pj = P[jl]
        jrels = ("cl", "ld", "we", "ex", "hd")
        tabs = {}
        for src_x, rels in ((hs, ("cl", "ld")), (hm, ("we", "ex")), (hr, ("hd",))):
            Wall = jnp.concatenate([_tabW(pj[r]) for r in rels], axis=1)
            out = _mm(src_x, Wall, None, False)
            for i, r in enumerate(rels):
                t = out[:, i * 160:(i + 1) * 160]
                tabs[r] = jnp.concatenate([t[:, :80], t[:, 80:]], axis=0)
        Wad_all = jnp.concatenate([_adW(pj[r]) for r in jrels], axis=1)
        adt_all = _mm(hj, Wad_all, None, False)
        outs = _sc_gat_multi(
            [(edges[r][0], edges[r][1], tabs[r],
              adt_all[:, i * 16:(i + 1) * 16], NJ)
             for i, r in enumerate(jrels)], od=DJ)
        parts = []
        for o2 in outs:
            parts += [o2[:NJ], o2[NJ:]]
        bsum = sum(pj[r]["b"] for r in jrels)
        hj = _combine(parts, hj, bsum, pj["ln_g"], pj["ln_b"], od=DJ)

        po = P[ol]
        orels = ("cbl", "li", "nd", "eb", "hb")
        Wall_o = jnp.concatenate([_tabW(po[r]) for r in orels], axis=1)
        outo = _mm(hj, Wall_o, None, False)
        otabs = {}
        for i, r in enumerate(orels):
            t = outo[:, i * 96:(i + 1) * 96]
            otabs[r] = jnp.concatenate([t[:, :48], t[:, 48:]], axis=0)
        adts = {}
        for xd, rels in ((hs, ("cbl", "li")), (hm, ("nd", "eb")), (hr, ("hb",))):
            Wad = jnp.concatenate([_adW(po[r]) for r in rels], axis=1)
            out = _mm(xd, Wad, None, False)
            for i, r in enumerate(rels):
                adts[r] = out[:, i * 16:(i + 1) * 16]
        ond = (("cbl", NS), ("li", NS), ("nd", NM), ("eb", NM), ("hb", NR))
        oouts = _sc_gat_multi(
            [(edges[r][0], edges[r][1], otabs[r], adts[r], nd) for r, nd in ond],
            od=DO)
        oparts = {}
        for (r, nd), o2 in zip(ond, oouts):
            oparts[r] = [o2[:nd], o2[nd:]]
        hs = _combine(oparts["cbl"] + oparts["li"], hs,
                      po["cbl"]["b"] + po["li"]["b"],
                      po["ln_gs"], po["ln_bs"], od=DO)
        hm = _combine(oparts["nd"] + oparts["eb"], hm,
                      po["nd"]["b"] + po["eb"]["b"],
                      po["ln_gm"], po["ln_bm"], od=DO)
        hr = _combine(oparts["hb"], hr, po["hb"]["b"],
                      po["ln_gr"], po["ln_br"], od=DO)

    hjg = hj.reshape(B, J * DJ)
    hn = jnp.concatenate([hs.reshape(B, 3 * DO), hm.reshape(B, 2 * DO), hr], axis=1)
    wg_big = jnp.kron(jnp.eye(J, dtype=jnp.float32), P["gate_W"])
    h_global = _pool_global(hjg, hn, wg_big, P["gl_W"], P["gl_b"])

    job_ids = actions[:, 0]
    graph = job_ids // J
    gji = job_ids + graph * J
    idx = jnp.concatenate([gji, NJ + graph]).astype(jnp.int32).reshape(64, 128)
    tab = jnp.concatenate([hj, h_global], axis=0)
    rowsg = _sc_gather(tab, idx)
    emb = rowsg[:A]
    hg = rowsg[A:]
    feat = jnp.concatenate(
        [emb, hg, actions[:, 1:2].astype(jnp.float32),
         actions[:, 2:3].astype(jnp.float32),
         jnp.broadcast_to(alpha.reshape(1, 1).astype(jnp.float32), (A, 1))],
        axis=1)
    return _head(feat, P)[:, 0]


# speedup vs baseline: 1.2088x; 1.2088x over previous
import functools

import jax
import jax.numpy as jnp
from jax import lax
from jax.experimental import pallas as pl
from jax.experimental.pallas import tpu as pltpu
from jax.experimental.pallas import tpu_sc as plsc

B = 1024
J = 16
NJ = B * J
NS = 3 * B
NM = 2 * B
NR = B
DJ = 128
DO = 64
H = 4
GD = 128
A = 4096


def _mm_body(x_ref, w_ref, b_ref, o_ref, *, act):
    y = jnp.dot(x_ref[...], w_ref[...], preferred_element_type=jnp.float32)
    y = y + b_ref[...]
    if act:
        y = jnp.maximum(y, 0.0)
    o_ref[...] = y


def _mm(x, W, b, act, bm=2048):
    M, K = x.shape
    N = W.shape[1]
    bvec = jnp.zeros((1, N), jnp.float32) if b is None else b.reshape(1, N)
    bm = min(bm, M)
    return pl.pallas_call(
        functools.partial(_mm_body, act=act),
        grid=(M // bm,),
        in_specs=[pl.BlockSpec((bm, K), lambda i: (i, 0)),
                  pl.BlockSpec((K, N), lambda i: (0, 0)),
                  pl.BlockSpec((1, N), lambda i: (0, 0))],
        out_specs=pl.BlockSpec((bm, N), lambda i: (i, 0)),
        out_shape=jax.ShapeDtypeStruct((M, N), jnp.float32),
    )(x, W, bvec)


def _sc_gat_multi(rels, od):
    od2 = od // 2
    TW = od2 + 16
    C = od // H
    KCH = 128
    NB = od2 // 16
    R = len(rels)
    nd_max = max(r[4] for r in rels)

    mesh = plsc.VectorSubcoreMesh(core_axis_name="c", subcore_axis_name="s")

    @functools.partial(
        pl.kernel, mesh=mesh,
        compiler_params=pltpu.CompilerParams(use_tc_tiling_on_sc=False),
        out_type=tuple(jax.ShapeDtypeStruct((2 * r[4], TW), jnp.float32)
                       for r in rels),
        scratch_types=[
            pltpu.VMEM((3, KCH), jnp.int32),
            pltpu.VMEM((3, KCH), jnp.int32),
            pltpu.VMEM((3, KCH, TW), jnp.float32),
            pltpu.VMEM((3, KCH, 16), jnp.float32),
            pltpu.VMEM((32, TW), jnp.float32),
            pltpu.VMEM_SHARED((nd_max, TW), jnp.float32),
        ] + [pltpu.SemaphoreType.DMA] * 6,
    )
    def k(*refs):
        ins = refs[:4 * R]
        outs = refs[4 * R:5 * R]
        isrc, idst, rows, adrows, zbuf, acc = refs[5 * R:5 * R + 6]
        semg = refs[5 * R + 6:5 * R + 9]
        semw = refs[5 * R + 9:5 * R + 12]
        c = lax.axis_index("c")
        s = lax.axis_index("s")

        def zrow(i, _):
            for bb in range(TW // 16):
                zbuf[i, pl.ds(bb * 16, 16)] = jnp.zeros((16,), jnp.float32)
            return 0
        lax.fori_loop(0, 32, zrow, 0)

        for r in range(R):
            src_h, dst_h, tab_h, adt_h = ins[4 * r:4 * r + 4]
            out_h = outs[r]
            E = src_h.shape[0]
            ns = tab_h.shape[0] // 2
            nd = rels[r][4]
            EPT = E // 16
            NCH = EPT // KCH
            RPT = nd // 16
            off = c * ns

            def zcp(i, _):
                pltpu.sync_copy(zbuf, acc.at[pl.ds(s * RPT + i * 32, 32)])
                return 0
            lax.fori_loop(0, RPT // 32, zcp, 0)
            plsc.subcore_barrier()

            def launch(i, p):
                base = s * EPT + i * KCH
                pltpu.sync_copy(src_h.at[pl.ds(base, KCH)], isrc.at[p])
                pltpu.sync_copy(dst_h.at[pl.ds(base, KCH)], idst.at[p])

                @plsc.parallel_loop(0, KCH // 16, 1, unroll=4)
                def addoff(j):
                    isrc[p, pl.ds(j * 16, 16)] = isrc[p, pl.ds(j * 16, 16)] + off
                pltpu.async_copy(tab_h.at[isrc.at[p]], rows.at[p], semg[p])
                pltpu.async_copy(adt_h.at[idst.at[p]], adrows.at[p], semg[p])

            def wait_scatter(p):
                pltpu.make_async_copy(rows.at[p], acc.at[idst.at[p]], semw[p]).wait()

            def process(i, p):
                pltpu.make_async_copy(tab_h.at[isrc.at[p]], rows.at[p], semg[p]).wait()
                pltpu.make_async_copy(adt_h.at[idst.at[p]], adrows.at[p], semg[p]).wait()

                @plsc.parallel_loop(0, KCH, 1, unroll=2)
                def edge(kk):
                    a_s = rows[p, kk, pl.ds(od2, 16)]
                    a_d = adrows[p, kk, :]
                    e = a_s + a_d
                    e = jnp.maximum(e, 0.2 * e)
                    w = jnp.exp(e)
                    rows[p, kk, pl.ds(od2, 16)] = w
                    w0 = jnp.where(c == 0, w[0], w[2])
                    w1 = jnp.where(c == 0, w[1], w[3])
                    for bb in range(NB):
                        ws = w0 if (bb * 16) // C == 0 else w1
                        rows[p, kk, pl.ds(bb * 16, 16)] = (
                            rows[p, kk, pl.ds(bb * 16, 16)] * ws)
                pltpu.async_copy(rows.at[p], acc.at[idst.at[p]], semw[p], add=True)

            def step(j, q):
                nb = (q + 1) % 3

                @pl.when(j >= 2)
                def _():
                    wait_scatter(nb)

                @pl.when(j + 1 < NCH)
                def _():
                    launch(j + 1, nb)
                process(j, q)

            launch(0, 0)

            def trio(i, _):
                for q in range(3):
                    step(3 * i + q, q)
                return 0
            lax.fori_loop(0, NCH // 3, trio, 0)
            for j in range(3 * (NCH // 3), NCH):
                step(jnp.int32(j), j % 3)
            for t in (NCH - 2, NCH - 1):
                wait_scatter(t % 3)
            plsc.subcore_barrier()
            pltpu.sync_copy(acc.at[pl.ds(s * RPT, RPT)],
                            out_h.at[pl.ds(c * nd + s * RPT, RPT)])

    flat = []
    for src, dst, tab, adt, _nd in rels:
        flat += [src, dst, tab, adt]
    return k(*flat)


def _sc_gather(tab, idx2d):
    D = tab.shape[1]
    mesh = plsc.VectorSubcoreMesh(core_axis_name="c", subcore_axis_name="s")

    @functools.partial(
        pl.kernel, mesh=mesh,
        compiler_params=pltpu.CompilerParams(use_tc_tiling_on_sc=False),
        out_type=jax.ShapeDtypeStruct((64 * 128, D), jnp.float32),
        scratch_types=[
            pltpu.VMEM((128,), jnp.int32),
            pltpu.VMEM((128, D), jnp.float32),
            pltpu.SemaphoreType.DMA,
        ],
    )
    def k(tab_h, idx_h, out_h, idxv, rows, sem):
        c = lax.axis_index("c")
        s = lax.axis_index("s")
        wid = s * 2 + c
        for j in range(2):
            pltpu.sync_copy(idx_h.at[wid * 2 + j], idxv)
            pltpu.async_copy(tab_h.at[idxv], rows, sem).wait()
            pltpu.sync_copy(rows, out_h.at[pl.ds((wid * 2 + j) * 128, 128)])

    return k(tab, idx2d)


def _combine_body(*refs, n_rel, od, bm):
    od2 = od // 2
    C = od // H
    part_refs = refs[:2 * n_rel]
    res_ref, bsum_ref, g_ref, b_ref, o_ref = refs[2 * n_rel:]
    cols = []
    for c in range(2):
        acc = jnp.zeros((bm, od2), jnp.float32)
        for r in range(n_rel):
            t = part_refs[2 * r + c][...]
            num = t[:, :od2]
            den = t[:, od2: od2 + 2]
            dene = jnp.concatenate(
                [jnp.broadcast_to(den[:, kk:kk + 1], (bm, C)) for kk in range(2)],
                axis=1)
            acc = acc + num / (dene + 1e-16)
        cols.append(acc)
    msg = jnp.concatenate(cols, axis=1) + bsum_ref[...]
    x = jnp.maximum(msg + res_ref[...], 0.0)
    mu = jnp.mean(x, axis=1, keepdims=True)
    xc = x - mu
    v = jnp.mean(xc * xc, axis=1, keepdims=True)
    o_ref[...] = xc / jnp.sqrt(v + 1e-5) * g_ref[...] + b_ref[...]


def _combine(parts, res, bsum, g, b, od, bm=1024):
    nd = res.shape[0]
    od2 = od // 2
    TW = od2 + 16
    n_rel = len(parts) // 2
    bm = min(bm, nd)
    pspec = [pl.BlockSpec((bm, TW), lambda i: (i, 0)) for _ in parts]
    return pl.pallas_call(
        functools.partial(_combine_body, n_rel=n_rel, od=od, bm=bm),
        grid=(nd // bm,),
        in_specs=pspec + [pl.BlockSpec((bm, od), lambda i: (i, 0)),
                          pl.BlockSpec((1, od), lambda i: (0, 0)),
                          pl.BlockSpec((1, od), lambda i: (0, 0)),
                          pl.BlockSpec((1, od), lambda i: (0, 0))],
        out_specs=pl.BlockSpec((bm, od), lambda i: (i, 0)),
        out_shape=jax.ShapeDtypeStruct((nd, od), jnp.float32),
    )(*parts, res, bsum.reshape(1, od), g.reshape(1, od), b.reshape(1, od))


def _pool_body(hjg_ref, hn_ref, wg_ref, glw_ref, glb_ref, o_ref, *, gb):
    hjg = hjg_ref[...]
    gate = jnp.dot(hjg, wg_ref[...], preferred_element_type=jnp.float32)
    gate = gate - jnp.max(gate, axis=1, keepdims=True)
    ge = jnp.exp(gate)
    w = ge / (jnp.sum(ge, axis=1, keepdims=True) + 1e-16)
    mj = jnp.zeros((gb, DJ), jnp.float32)
    for j in range(J):
        mj = mj + w[:, j:j + 1] * hjg[:, j * DJ:(j + 1) * DJ]
    feat = jnp.concatenate([hn_ref[...], mj], axis=1)
    y = jnp.dot(feat, glw_ref[...], preferred_element_type=jnp.float32)
    o_ref[...] = jnp.maximum(y + glb_ref[...], 0.0)


def _pool_global(hjg, hn, wg_big, gl_W, gl_b, gb=256):
    return pl.pallas_call(
        functools.partial(_pool_body, gb=gb),
        grid=(B // gb,),
        in_specs=[pl.BlockSpec((gb, J * DJ), lambda i: (i, 0)),
                  pl.BlockSpec((gb, 6 * DO), lambda i: (i, 0)),
                  pl.BlockSpec((J * DJ, J), lambda i: (0, 0)),
                  pl.BlockSpec((6 * DO + DJ, GD), lambda i: (0, 0)),
                  pl.BlockSpec((1, GD), lambda i: (0, 0))],
        out_specs=pl.BlockSpec((gb, GD), lambda i: (i, 0)),
        out_shape=jax.ShapeDtypeStruct((B, GD), jnp.float32),
    )(hjg, hn, wg_big, gl_W, gl_b.reshape(1, GD))


def _head_body(f_ref, w1, b1, w2, b2, w3, b3, o_ref):
    h = jnp.dot(f_ref[...], w1[...], preferred_element_type=jnp.float32) + b1[...]
    h = jnp.maximum(h, 0.0)
    h = jnp.dot(h, w2[...], preferred_element_type=jnp.float32) + b2[...]
    h = jnp.maximum(h, 0.0)
    o_ref[...] = jnp.dot(h, w3[...], preferred_element_type=jnp.float32) + b3[...]


def _head(feat, P, bm=2048):
    K = feat.shape[1]
    return pl.pallas_call(
        _head_body,
        grid=(A // bm,),
        in_specs=[pl.BlockSpec((bm, K), lambda i: (i, 0)),
                  pl.BlockSpec((K, 64), lambda i: (0, 0)),
                  pl.BlockSpec((1, 64), lambda i: (0, 0)),
                  pl.BlockSpec((64, 32), lambda i: (0, 0)),
                  pl.BlockSpec((1, 32), lambda i: (0, 0)),
                  pl.BlockSpec((32, 1), lambda i: (0, 0)),
                  pl.BlockSpec((1, 1), lambda i: (0, 0))],
        out_specs=pl.BlockSpec((bm, 1), lambda i: (i, 0)),
        out_shape=jax.ShapeDtypeStruct((A, 1), jnp.float32),
    )(feat, P["q1_W"], P["q1_b"].reshape(1, 64), P["q2_W"],
      P["q2_b"].reshape(1, 32), P["q3_W"], P["q3_b"].reshape(1, 1))


def _fold_attn(W, avec):
    sd, od = W.shape
    C = od // H
    return (W.reshape(sd, H, C) * avec[None]).sum(-1)


def _tabW(p):
    Ws = p["Ws"]
    od = p["b"].shape[0]
    od2 = od // 2
    Wa = jnp.pad(_fold_attn(Ws, p["as"]), ((0, 0), (0, 12)))
    return jnp.concatenate([Ws[:, :od2], Wa, Ws[:, od2:], Wa], axis=1)


def _adW(p):
    return jnp.pad(_fold_attn(p["Wd"], p["ad"]), ((0, 0), (0, 12)))


def kernel(x_job, x_station, x_machine, x_robot, alpha, actions, params, edges):
    P = params
    hj = _mm(x_job, P["lj_W"], P["lj_b"], True)
    hs = _mm(x_station, P["ls_W"], P["ls_b"], True)
    hm = _mm(x_machine, P["lm_W"], P["lm_b"], True)
    hr = _mm(x_robot, P["lr_W"], P["lr_b"], True)

    for jl, ol in (("j1", "o1"), ("j2", "o2")):
        pj = P[jl]
        jrels = ("cl", "ld", "we", "ex", "hd")
        tabs = {}
        for src_x, rels in ((hs, ("cl", "ld")), (hm, ("we", "ex")), (hr, ("hd",))):
            Wall = jnp.concatenate([_tabW(pj[r]) for r in rels], axis=1)
            out = _mm(src_x, Wall, None, False)
            for i, r in enumerate(rels):
                t = out[:, i * 160:(i + 1) * 160]
                tabs[r] = jnp.concatenate([t[:, :80], t[:, 80:]], axis=0)
        Wad_all = jnp.concatenate([_adW(pj[r]) for r in jrels], axis=1)
        adt_all = _mm(hj, Wad_all, None, False)
        parts = []
        for i, r in enumerate(jrels):
            o2, = _sc_gat_multi(
                [(edges[r][0], edges[r][1], tabs[r],
                  adt_all[:, i * 16:(i + 1) * 16], NJ)], od=DJ)
            parts += [o2[:NJ], o2[NJ:]]
        bsum = sum(pj[r]["b"] for r in jrels)
        hj = _combine(parts, hj, bsum, pj["ln_g"], pj["ln_b"], od=DJ)

        po = P[ol]
        orels = ("cbl", "li", "nd", "eb", "hb")
        Wall_o = jnp.concatenate([_tabW(po[r]) for r in orels], axis=1)
        outo = _mm(hj, Wall_o, None, False)
        otabs = {}
        for i, r in enumerate(orels):
            t = outo[:, i * 96:(i + 1) * 96]
            otabs[r] = jnp.concatenate([t[:, :48], t[:, 48:]], axis=0)
        adts = {}
        for xd, rels in ((hs, ("cbl", "li")), (hm, ("nd", "eb")), (hr, ("hb",))):
            Wad = jnp.concatenate([_adW(po[r]) for r in rels], axis=1)
            out = _mm(xd, Wad, None, False)
            for i, r in enumerate(rels):
                adts[r] = out[:, i * 16:(i + 1) * 16]
        oparts = {}
        for r, nd in (("cbl", NS), ("li", NS), ("nd", NM), ("eb", NM), ("hb", NR)):
            o2, = _sc_gat_multi(
                [(edges[r][0], edges[r][1], otabs[r], adts[r], nd)], od=DO)
            oparts[r] = [o2[:nd], o2[nd:]]
        hs = _combine(oparts["cbl"] + oparts["li"], hs,
                      po["cbl"]["b"] + po["li"]["b"],
                      po["ln_gs"], po["ln_bs"], od=DO)
        hm = _combine(oparts["nd"] + oparts["eb"], hm,
                      po["nd"]["b"] + po["eb"]["b"],
                      po["ln_gm"], po["ln_bm"], od=DO)
        hr = _combine(oparts["hb"], hr, po["hb"]["b"],
                      po["ln_gr"], po["ln_br"], od=DO)

    hjg = hj.reshape(B, J * DJ)
    hn = jnp.concatenate([hs.reshape(B, 3 * DO), hm.reshape(B, 2 * DO), hr], axis=1)
    wg_big = jnp.kron(jnp.eye(J, dtype=jnp.float32), P["gate_W"])
    h_global = _pool_global(hjg, hn, wg_big, P["gl_W"], P["gl_b"])

    job_ids = actions[:, 0]
    graph = job_ids // J
    gji = job_ids + graph * J
    idx = jnp.concatenate([gji, NJ + graph]).astype(jnp.int32).reshape(64, 128)
    tab = jnp.concatenate([hj, h_global], axis=0)
    rowsg = _sc_gather(tab, idx)
    emb = rowsg[:A]
    hg = rowsg[A:]
    feat = jnp.concatenate(
        [emb, hg, actions[:, 1:2].astype(jnp.float32),
         actions[:, 2:3].astype(jnp.float32),
         jnp.broadcast_to(alpha.reshape(1, 1).astype(jnp.float32), (A, 1))],
        axis=1)
    return _head(feat, P)[:, 0]


# speedup vs baseline: 1.3640x; 1.1284x over previous
import functools

import jax
import jax.numpy as jnp
from jax import lax
from jax.experimental import pallas as pl
from jax.experimental.pallas import tpu as pltpu
from jax.experimental.pallas import tpu_sc as plsc

B = 1024
J = 16
NJ = B * J
NS = 3 * B
NM = 2 * B
NR = B
DJ = 128
DO = 64
H = 4
GD = 128
A = 4096


def _mm_body(x_ref, w_ref, b_ref, o_ref, *, act):
    y = jnp.dot(x_ref[...], w_ref[...], preferred_element_type=jnp.float32)
    y = y + b_ref[...]
    if act:
        y = jnp.maximum(y, 0.0)
    o_ref[...] = y


def _mm(x, W, b, act, bm=2048):
    M, K = x.shape
    N = W.shape[1]
    bvec = jnp.zeros((1, N), jnp.float32) if b is None else b.reshape(1, N)
    bm = min(bm, M)
    return pl.pallas_call(
        functools.partial(_mm_body, act=act),
        grid=(M // bm,),
        in_specs=[pl.BlockSpec((bm, K), lambda i: (i, 0)),
                  pl.BlockSpec((K, N), lambda i: (0, 0)),
                  pl.BlockSpec((1, N), lambda i: (0, 0))],
        out_specs=pl.BlockSpec((bm, N), lambda i: (i, 0)),
        out_shape=jax.ShapeDtypeStruct((M, N), jnp.float32),
    )(x, W, bvec)


def _sc_gat_multi(rels, od):
    od2 = od // 2
    TW = od2 + 16
    C = od // H
    KCH = 128
    NB = od2 // 16
    R = len(rels)
    nd_max = max(r[4] for r in rels)
    NIR = max(r[0].shape[0] // 16 for r in rels)

    mesh = plsc.VectorSubcoreMesh(core_axis_name="c", subcore_axis_name="s")

    @functools.partial(
        pl.kernel, mesh=mesh,
        compiler_params=pltpu.CompilerParams(use_tc_tiling_on_sc=False),
        out_type=tuple(jax.ShapeDtypeStruct((2 * r[4], TW), jnp.float32)
                       for r in rels),
        scratch_types=[
            pltpu.VMEM((NIR, KCH), jnp.int32),
            pltpu.VMEM((NIR, KCH), jnp.int32),
            pltpu.VMEM((3, KCH, TW), jnp.float32),
            pltpu.VMEM((3, KCH, 16), jnp.float32),
            pltpu.VMEM((32, TW), jnp.float32),
            pltpu.VMEM_SHARED((nd_max, TW), jnp.float32),
        ] + [pltpu.SemaphoreType.DMA] * 6,
    )
    def k(*refs):
        ins = refs[:4 * R]
        outs = refs[4 * R:5 * R]
        isrc, idst, rows, adrows, zbuf, acc = refs[5 * R:5 * R + 6]
        semg = refs[5 * R + 6:5 * R + 9]
        semw = refs[5 * R + 9:5 * R + 12]
        c = lax.axis_index("c")
        s = lax.axis_index("s")

        def zrow(i, _):
            for bb in range(TW // 16):
                zbuf[i, pl.ds(bb * 16, 16)] = jnp.zeros((16,), jnp.float32)
            return 0
        lax.fori_loop(0, 32, zrow, 0)

        for r in range(R):
            src_h, dst_h, tab_h, adt_h = ins[4 * r:4 * r + 4]
            out_h = outs[r]
            E = src_h.shape[0] * KCH
            ns = tab_h.shape[0] // 2
            nd = rels[r][4]
            EPT = E // 16
            NCH = EPT // KCH
            RPT = nd // 16
            off = c * ns

            pltpu.sync_copy(src_h.at[pl.ds(s * NCH, NCH)], isrc.at[pl.ds(0, NCH)])
            pltpu.sync_copy(dst_h.at[pl.ds(s * NCH, NCH)], idst.at[pl.ds(0, NCH)])

            def offrow(i, _):
                @plsc.parallel_loop(0, KCH // 16, 1, unroll=4)
                def addoff(j):
                    isrc[i, pl.ds(j * 16, 16)] = isrc[i, pl.ds(j * 16, 16)] + off
                return 0
            lax.fori_loop(0, NCH, offrow, 0)

            def zcp(i, _):
                pltpu.sync_copy(zbuf, acc.at[pl.ds(s * RPT + i * 32, 32)])
                return 0
            lax.fori_loop(0, RPT // 32, zcp, 0)
            plsc.subcore_barrier()

            def launch(i, p):
                pltpu.async_copy(tab_h.at[isrc.at[i]], rows.at[p], semg[p])
                pltpu.async_copy(adt_h.at[idst.at[i]], adrows.at[p], semg[p])

            def wait_scatter(p):
                pltpu.make_async_copy(rows.at[p], acc.at[idst.at[0]],
                                      semw[p]).wait()

            def process(i, p):
                pltpu.make_async_copy(tab_h.at[isrc.at[0]], rows.at[p],
                                      semg[p]).wait()
                pltpu.make_async_copy(adt_h.at[idst.at[0]], adrows.at[p],
                                      semg[p]).wait()

                @plsc.parallel_loop(0, KCH, 1, unroll=2)
                def edge(kk):
                    a_s = rows[p, kk, pl.ds(od2, 16)]
                    a_d = adrows[p, kk, :]
                    e = a_s + a_d
                    e = jnp.maximum(e, 0.2 * e)
                    w = jnp.exp(e)
                    rows[p, kk, pl.ds(od2, 16)] = w
                    w0 = jnp.where(c == 0, w[0], w[2])
                    w1 = jnp.where(c == 0, w[1], w[3])
                    for bb in range(NB):
                        ws = w0 if (bb * 16) // C == 0 else w1
                        rows[p, kk, pl.ds(bb * 16, 16)] = (
                            rows[p, kk, pl.ds(bb * 16, 16)] * ws)
                pltpu.async_copy(rows.at[p], acc.at[idst.at[i]], semw[p], add=True)

            def step(j, q):
                nb = (q + 1) % 3

                @pl.when(j >= 2)
                def _():
                    wait_scatter(nb)

                @pl.when(j + 1 < NCH)
                def _():
                    launch(j + 1, nb)
                process(j, q)

            launch(0, 0)

            def trio(i, _):
                for q in range(3):
                    step(3 * i + q, q)
                return 0
            lax.fori_loop(0, NCH // 3, trio, 0)
            for j in range(3 * (NCH // 3), NCH):
                step(jnp.int32(j), j % 3)
            for t in (NCH - 2, NCH - 1):
                wait_scatter(t % 3)
            plsc.subcore_barrier()
            pltpu.sync_copy(acc.at[pl.ds(s * RPT, RPT)],
                            out_h.at[pl.ds(c * nd + s * RPT, RPT)])

    flat = []
    for src, dst, tab, adt, _nd in rels:
        flat += [src, dst, tab, adt]
    return k(*flat)


def _sc_gather(tab, idx2d):
    D = tab.shape[1]
    mesh = plsc.VectorSubcoreMesh(core_axis_name="c", subcore_axis_name="s")

    @functools.partial(
        pl.kernel, mesh=mesh,
        compiler_params=pltpu.CompilerParams(use_tc_tiling_on_sc=False),
        out_type=jax.ShapeDtypeStruct((64 * 128, D), jnp.float32),
        scratch_types=[
            pltpu.VMEM((128,), jnp.int32),
            pltpu.VMEM((128, D), jnp.float32),
            pltpu.SemaphoreType.DMA,
        ],
    )
    def k(tab_h, idx_h, out_h, idxv, rows, sem):
        c = lax.axis_index("c")
        s = lax.axis_index("s")
        wid = s * 2 + c
        for j in range(2):
            pltpu.sync_copy(idx_h.at[wid * 2 + j], idxv)
            pltpu.async_copy(tab_h.at[idxv], rows, sem).wait()
            pltpu.sync_copy(rows, out_h.at[pl.ds((wid * 2 + j) * 128, 128)])

    return k(tab, idx2d)


def _combine_body(*refs, n_rel, od, bm):
    od2 = od // 2
    C = od // H
    part_refs = refs[:2 * n_rel]
    res_ref, bsum_ref, g_ref, b_ref, o_ref = refs[2 * n_rel:]
    cols = []
    for c in range(2):
        acc = jnp.zeros((bm, od2), jnp.float32)
        for r in range(n_rel):
            t = part_refs[2 * r + c][...]
            num = t[:, :od2]
            den = t[:, od2: od2 + 2]
            dene = jnp.concatenate(
                [jnp.broadcast_to(den[:, kk:kk + 1], (bm, C)) for kk in range(2)],
                axis=1)
            acc = acc + num / (dene + 1e-16)
        cols.append(acc)
    msg = jnp.concatenate(cols, axis=1) + bsum_ref[...]
    x = jnp.maximum(msg + res_ref[...], 0.0)
    mu = jnp.mean(x, axis=1, keepdims=True)
    xc = x - mu
    v = jnp.mean(xc * xc, axis=1, keepdims=True)
    o_ref[...] = xc / jnp.sqrt(v + 1e-5) * g_ref[...] + b_ref[...]


def _combine(parts, res, bsum, g, b, od, bm=1024):
    nd = res.shape[0]
    od2 = od // 2
    TW = od2 + 16
    n_rel = len(parts) // 2
    bm = min(bm, nd)
    pspec = [pl.BlockSpec((bm, TW), lambda i: (i, 0)) for _ in parts]
    return pl.pallas_call(
        functools.partial(_combine_body, n_rel=n_rel, od=od, bm=bm),
        grid=(nd // bm,),
        in_specs=pspec + [pl.BlockSpec((bm, od), lambda i: (i, 0)),
                          pl.BlockSpec((1, od), lambda i: (0, 0)),
                          pl.BlockSpec((1, od), lambda i: (0, 0)),
                          pl.BlockSpec((1, od), lambda i: (0, 0))],
        out_specs=pl.BlockSpec((bm, od), lambda i: (i, 0)),
        out_shape=jax.ShapeDtypeStruct((nd, od), jnp.float32),
    )(*parts, res, bsum.reshape(1, od), g.reshape(1, od), b.reshape(1, od))


def _pool_body(hjg_ref, hn_ref, wg_ref, glw_ref, glb_ref, o_ref, *, gb):
    hjg = hjg_ref[...]
    gate = jnp.dot(hjg, wg_ref[...], preferred_element_type=jnp.float32)
    gate = gate - jnp.max(gate, axis=1, keepdims=True)
    ge = jnp.exp(gate)
    w = ge / (jnp.sum(ge, axis=1, keepdims=True) + 1e-16)
    mj = jnp.zeros((gb, DJ), jnp.float32)
    for j in range(J):
        mj = mj + w[:, j:j + 1] * hjg[:, j * DJ:(j + 1) * DJ]
    feat = jnp.concatenate([hn_ref[...], mj], axis=1)
    y = jnp.dot(feat, glw_ref[...], preferred_element_type=jnp.float32)
    o_ref[...] = jnp.maximum(y + glb_ref[...], 0.0)


def _pool_global(hjg, hn, wg_big, gl_W, gl_b, gb=256):
    return pl.pallas_call(
        functools.partial(_pool_body, gb=gb),
        grid=(B // gb,),
        in_specs=[pl.BlockSpec((gb, J * DJ), lambda i: (i, 0)),
                  pl.BlockSpec((gb, 6 * DO), lambda i: (i, 0)),
                  pl.BlockSpec((J * DJ, J), lambda i: (0, 0)),
                  pl.BlockSpec((6 * DO + DJ, GD), lambda i: (0, 0)),
                  pl.BlockSpec((1, GD), lambda i: (0, 0))],
        out_specs=pl.BlockSpec((gb, GD), lambda i: (i, 0)),
        out_shape=jax.ShapeDtypeStruct((B, GD), jnp.float32),
    )(hjg, hn, wg_big, gl_W, gl_b.reshape(1, GD))


def _head_body(f_ref, w1, b1, w2, b2, w3, b3, o_ref):
    h = jnp.dot(f_ref[...], w1[...], preferred_element_type=jnp.float32) + b1[...]
    h = jnp.maximum(h, 0.0)
    h = jnp.dot(h, w2[...], preferred_element_type=jnp.float32) + b2[...]
    h = jnp.maximum(h, 0.0)
    o_ref[...] = jnp.dot(h, w3[...], preferred_element_type=jnp.float32) + b3[...]


def _head(feat, P, bm=2048):
    K = feat.shape[1]
    return pl.pallas_call(
        _head_body,
        grid=(A // bm,),
        in_specs=[pl.BlockSpec((bm, K), lambda i: (i, 0)),
                  pl.BlockSpec((K, 64), lambda i: (0, 0)),
                  pl.BlockSpec((1, 64), lambda i: (0, 0)),
                  pl.BlockSpec((64, 32), lambda i: (0, 0)),
                  pl.BlockSpec((1, 32), lambda i: (0, 0)),
                  pl.BlockSpec((32, 1), lambda i: (0, 0)),
                  pl.BlockSpec((1, 1), lambda i: (0, 0))],
        out_specs=pl.BlockSpec((bm, 1), lambda i: (i, 0)),
        out_shape=jax.ShapeDtypeStruct((A, 1), jnp.float32),
    )(feat, P["q1_W"], P["q1_b"].reshape(1, 64), P["q2_W"],
      P["q2_b"].reshape(1, 32), P["q3_W"], P["q3_b"].reshape(1, 1))


def _fold_attn(W, avec):
    sd, od = W.shape
    C = od // H
    return (W.reshape(sd, H, C) * avec[None]).sum(-1)


def _tabW(p):
    Ws = p["Ws"]
    od = p["b"].shape[0]
    od2 = od // 2
    Wa = jnp.pad(_fold_attn(Ws, p["as"]), ((0, 0), (0, 12)))
    return jnp.concatenate([Ws[:, :od2], Wa, Ws[:, od2:], Wa], axis=1)


def _adW(p):
    return jnp.pad(_fold_attn(p["Wd"], p["ad"]), ((0, 0), (0, 12)))


def kernel(x_job, x_station, x_machine, x_robot, alpha, actions, params, edges):
    P = params
    hj = _mm(x_job, P["lj_W"], P["lj_b"], True)
    hs = _mm(x_station, P["ls_W"], P["ls_b"], True)
    hm = _mm(x_machine, P["lm_W"], P["lm_b"], True)
    hr = _mm(x_robot, P["lr_W"], P["lr_b"], True)

    for jl, ol in (("j1", "o1"), ("j2", "o2")):
        pj = P[jl]
        jrels = ("cl", "ld", "we", "ex", "hd")
        tabs = {}
        for src_x, rels in ((hs, ("cl", "ld")), (hm, ("we", "ex")), (hr, ("hd",))):
            Wall = jnp.concatenate([_tabW(pj[r]) for r in rels], axis=1)
            out = _mm(src_x, Wall, None, False)
            for i, r in enumerate(rels):
                t = out[:, i * 160:(i + 1) * 160]
                tabs[r] = jnp.concatenate([t[:, :80], t[:, 80:]], axis=0)
        Wad_all = jnp.concatenate([_adW(pj[r]) for r in jrels], axis=1)
        adt_all = _mm(hj, Wad_all, None, False)
        parts = []
        for i, r in enumerate(jrels):
            o2, = _sc_gat_multi(
                [(edges[r][0].reshape(-1, 128), edges[r][1].reshape(-1, 128),
                  tabs[r], adt_all[:, i * 16:(i + 1) * 16], NJ)], od=DJ)
            parts += [o2[:NJ], o2[NJ:]]
        bsum = sum(pj[r]["b"] for r in jrels)
        hj = _combine(parts, hj, bsum, pj["ln_g"], pj["ln_b"], od=DJ)

        po = P[ol]
        orels = ("cbl", "li", "nd", "eb", "hb")
        Wall_o = jnp.concatenate([_tabW(po[r]) for r in orels], axis=1)
        outo = _mm(hj, Wall_o, None, False)
        otabs = {}
        for i, r in enumerate(orels):
            t = outo[:, i * 96:(i + 1) * 96]
            otabs[r] = jnp.concatenate([t[:, :48], t[:, 48:]], axis=0)
        adts = {}
        for xd, rels in ((hs, ("cbl", "li")), (hm, ("nd", "eb")), (hr, ("hb",))):
            Wad = jnp.concatenate([_adW(po[r]) for r in rels], axis=1)
            out = _mm(xd, Wad, None, False)
            for i, r in enumerate(rels):
                adts[r] = out[:, i * 16:(i + 1) * 16]
        oparts = {}
        for r, nd in (("cbl", NS), ("li", NS), ("nd", NM), ("eb", NM), ("hb", NR)):
            o2, = _sc_gat_multi(
                [(edges[r][0].reshape(-1, 128), edges[r][1].reshape(-1, 128),
                  otabs[r], adts[r], nd)], od=DO)
            oparts[r] = [o2[:nd], o2[nd:]]
        hs = _combine(oparts["cbl"] + oparts["li"], hs,
                      po["cbl"]["b"] + po["li"]["b"],
                      po["ln_gs"], po["ln_bs"], od=DO)
        hm = _combine(oparts["nd"] + oparts["eb"], hm,
                      po["nd"]["b"] + po["eb"]["b"],
                      po["ln_gm"], po["ln_bm"], od=DO)
        hr = _combine(oparts["hb"], hr, po["hb"]["b"],
                      po["ln_gr"], po["ln_br"], od=DO)

    hjg = hj.reshape(B, J * DJ)
    hn = jnp.concatenate([hs.reshape(B, 3 * DO), hm.reshape(B, 2 * DO), hr], axis=1)
    wg_big = jnp.kron(jnp.eye(J, dtype=jnp.float32), P["gate_W"])
    h_global = _pool_global(hjg, hn, wg_big, P["gl_W"], P["gl_b"])

    job_ids = actions[:, 0]
    graph = job_ids // J
    gji = job_ids + graph * J
    idx = jnp.concatenate([gji, NJ + graph]).astype(jnp.int32).reshape(64, 128)
    tab = jnp.concatenate([hj, h_global], axis=0)
    rowsg = _sc_gather(tab, idx)
    emb = rowsg[:A]
    hg = rowsg[A:]
    feat = jnp.concatenate(
        [emb, hg, actions[:, 1:2].astype(jnp.float32),
         actions[:, 2:3].astype(jnp.float32),
         jnp.broadcast_to(alpha.reshape(1, 1).astype(jnp.float32), (A, 1))],
        axis=1)
    return _head(feat, P)[:, 0]


# speedup vs baseline: 1.3690x; 1.0036x over previous
import functools

import jax
import jax.numpy as jnp
from jax import lax
from jax.experimental import pallas as pl
from jax.experimental.pallas import tpu as pltpu
from jax.experimental.pallas import tpu_sc as plsc

B = 1024
J = 16
NJ = B * J
NS = 3 * B
NM = 2 * B
NR = B
DJ = 128
DO = 64
H = 4
GD = 128
A = 4096


def _mm_body(x_ref, w_ref, b_ref, o_ref, *, act):
    y = jnp.dot(x_ref[...], w_ref[...], preferred_element_type=jnp.float32)
    y = y + b_ref[...]
    if act:
        y = jnp.maximum(y, 0.0)
    o_ref[...] = y


def _mm(x, W, b, act, bm=2048):
    M, K = x.shape
    N = W.shape[1]
    bvec = jnp.zeros((1, N), jnp.float32) if b is None else b.reshape(1, N)
    bm = min(bm, M)
    return pl.pallas_call(
        functools.partial(_mm_body, act=act),
        grid=(M // bm,),
        in_specs=[pl.BlockSpec((bm, K), lambda i: (i, 0)),
                  pl.BlockSpec((K, N), lambda i: (0, 0)),
                  pl.BlockSpec((1, N), lambda i: (0, 0))],
        out_specs=pl.BlockSpec((bm, N), lambda i: (i, 0)),
        out_shape=jax.ShapeDtypeStruct((M, N), jnp.float32),
    )(x, W, bvec)


def _sc_gat_multi(rels, od):
    od2 = od // 2
    TW = od2 + 16
    C = od // H
    KCH = 128
    NB = od2 // 16
    R = len(rels)
    nd_max = max(r[4] for r in rels)
    NIR = max(r[0].shape[0] // 16 for r in rels)

    mesh = plsc.VectorSubcoreMesh(core_axis_name="c", subcore_axis_name="s")

    @functools.partial(
        pl.kernel, mesh=mesh,
        compiler_params=pltpu.CompilerParams(use_tc_tiling_on_sc=False),
        out_type=tuple(jax.ShapeDtypeStruct((2 * r[4], TW), jnp.float32)
                       for r in rels),
        scratch_types=[
            pltpu.VMEM((NIR, KCH), jnp.int32),
            pltpu.VMEM((NIR, KCH), jnp.int32),
            pltpu.VMEM((3, KCH, TW), jnp.float32),
            pltpu.VMEM((3, KCH, 16), jnp.float32),
            pltpu.VMEM((32, TW), jnp.float32),
            pltpu.VMEM_SHARED((nd_max, TW), jnp.float32),
        ] + [pltpu.SemaphoreType.DMA] * 6,
    )
    def k(*refs):
        ins = refs[:4 * R]
        outs = refs[4 * R:5 * R]
        isrc, idst, rows, adrows, zbuf, acc = refs[5 * R:5 * R + 6]
        semg = refs[5 * R + 6:5 * R + 9]
        semw = refs[5 * R + 9:5 * R + 12]
        c = lax.axis_index("c")
        s = lax.axis_index("s")

        def zrow(i, _):
            for bb in range(TW // 16):
                zbuf[i, pl.ds(bb * 16, 16)] = jnp.zeros((16,), jnp.float32)
            return 0
        lax.fori_loop(0, 32, zrow, 0)

        for r in range(R):
            src_h, dst_h, tab_h, adt_h = ins[4 * r:4 * r + 4]
            out_h = outs[r]
            E = src_h.shape[0] * KCH
            ns = tab_h.shape[0] // 2
            nd = rels[r][4]
            EPT = E // 16
            NCH = EPT // KCH
            RPT = nd // 16
            off = c * ns

            pltpu.sync_copy(src_h.at[pl.ds(s * NCH, NCH)], isrc.at[pl.ds(0, NCH)])
            pltpu.sync_copy(dst_h.at[pl.ds(s * NCH, NCH)], idst.at[pl.ds(0, NCH)])

            def offrow(i, _):
                @plsc.parallel_loop(0, KCH // 16, 1, unroll=4)
                def addoff(j):
                    isrc[i, pl.ds(j * 16, 16)] = isrc[i, pl.ds(j * 16, 16)] + off
                return 0
            lax.fori_loop(0, NCH, offrow, 0)

            def zcp(i, _):
                pltpu.sync_copy(zbuf, acc.at[pl.ds(s * RPT + i * 32, 32)])
                return 0
            lax.fori_loop(0, RPT // 32, zcp, 0)
            plsc.subcore_barrier()

            def launch(i, p):
                pltpu.async_copy(tab_h.at[isrc.at[i]], rows.at[p], semg[p])
                pltpu.async_copy(adt_h.at[idst.at[i]], adrows.at[p], semg[p])

            def wait_scatter(p):
                pltpu.make_async_copy(rows.at[p], acc.at[idst.at[0]],
                                      semw[p]).wait()

            def process(i, p):
                pltpu.make_async_copy(tab_h.at[isrc.at[0]], rows.at[p],
                                      semg[p]).wait()
                pltpu.make_async_copy(adt_h.at[idst.at[0]], adrows.at[p],
                                      semg[p]).wait()

                @plsc.parallel_loop(0, KCH, 1, unroll=4)
                def edge(kk):
                    a_s = rows[p, kk, pl.ds(od2, 16)]
                    a_d = adrows[p, kk, :]
                    e = a_s + a_d
                    e = jnp.maximum(e, 0.2 * e)
                    w = jnp.exp(e)
                    rows[p, kk, pl.ds(od2, 16)] = w
                    w0 = jnp.where(c == 0, w[0], w[2])
                    w1 = jnp.where(c == 0, w[1], w[3])
                    for bb in range(NB):
                        ws = w0 if (bb * 16) // C == 0 else w1
                        rows[p, kk, pl.ds(bb * 16, 16)] = (
                            rows[p, kk, pl.ds(bb * 16, 16)] * ws)
                pltpu.async_copy(rows.at[p], acc.at[idst.at[i]], semw[p], add=True)

            def step(j, q):
                nb = (q + 1) % 3

                @pl.when(j >= 2)
                def _():
                    wait_scatter(nb)

                @pl.when(j + 1 < NCH)
                def _():
                    launch(j + 1, nb)
                process(j, q)

            launch(0, 0)

            def trio(i, _):
                for q in range(3):
                    step(3 * i + q, q)
                return 0
            lax.fori_loop(0, NCH // 3, trio, 0)
            for j in range(3 * (NCH // 3), NCH):
                step(jnp.int32(j), j % 3)
            for t in (NCH - 2, NCH - 1):
                wait_scatter(t % 3)
            plsc.subcore_barrier()
            pltpu.sync_copy(acc.at[pl.ds(s * RPT, RPT)],
                            out_h.at[pl.ds(c * nd + s * RPT, RPT)])

    flat = []
    for src, dst, tab, adt, _nd in rels:
        flat += [src, dst, tab, adt]
    return k(*flat)


def _sc_gather(tab, idx2d):
    D = tab.shape[1]
    mesh = plsc.VectorSubcoreMesh(core_axis_name="c", subcore_axis_name="s")

    @functools.partial(
        pl.kernel, mesh=mesh,
        compiler_params=pltpu.CompilerParams(use_tc_tiling_on_sc=False),
        out_type=jax.ShapeDtypeStruct((64 * 128, D), jnp.float32),
        scratch_types=[
            pltpu.VMEM((128,), jnp.int32),
            pltpu.VMEM((128, D), jnp.float32),
            pltpu.SemaphoreType.DMA,
        ],
    )
    def k(tab_h, idx_h, out_h, idxv, rows, sem):
        c = lax.axis_index("c")
        s = lax.axis_index("s")
        wid = s * 2 + c
        for j in range(2):
            pltpu.sync_copy(idx_h.at[wid * 2 + j], idxv)
            pltpu.async_copy(tab_h.at[idxv], rows, sem).wait()
            pltpu.sync_copy(rows, out_h.at[pl.ds((wid * 2 + j) * 128, 128)])

    return k(tab, idx2d)


def _combine_body(*refs, n_rel, od, bm):
    od2 = od // 2
    C = od // H
    part_refs = refs[:2 * n_rel]
    res_ref, bsum_ref, g_ref, b_ref, o_ref = refs[2 * n_rel:]
    cols = []
    for c in range(2):
        acc = jnp.zeros((bm, od2), jnp.float32)
        for r in range(n_rel):
            t = part_refs[2 * r + c][...]
            num = t[:, :od2]
            den = t[:, od2: od2 + 2]
            dene = jnp.concatenate(
                [jnp.broadcast_to(den[:, kk:kk + 1], (bm, C)) for kk in range(2)],
                axis=1)
            acc = acc + num / (dene + 1e-16)
        cols.append(acc)
    msg = jnp.concatenate(cols, axis=1) + bsum_ref[...]
    x = jnp.maximum(msg + res_ref[...], 0.0)
    mu = jnp.mean(x, axis=1, keepdims=True)
    xc = x - mu
    v = jnp.mean(xc * xc, axis=1, keepdims=True)
    o_ref[...] = xc / jnp.sqrt(v + 1e-5) * g_ref[...] + b_ref[...]


def _combine(parts, res, bsum, g, b, od, bm=1024):
    nd = res.shape[0]
    od2 = od // 2
    TW = od2 + 16
    n_rel = len(parts) // 2
    bm = min(bm, nd)
    pspec = [pl.BlockSpec((bm, TW), lambda i: (i, 0)) for _ in parts]
    return pl.pallas_call(
        functools.partial(_combine_body, n_rel=n_rel, od=od, bm=bm),
        grid=(nd // bm,),
        in_specs=pspec + [pl.BlockSpec((bm, od), lambda i: (i, 0)),
                          pl.BlockSpec((1, od), lambda i: (0, 0)),
                          pl.BlockSpec((1, od), lambda i: (0, 0)),
                          pl.BlockSpec((1, od), lambda i: (0, 0))],
        out_specs=pl.BlockSpec((bm, od), lambda i: (i, 0)),
        out_shape=jax.ShapeDtypeStruct((nd, od), jnp.float32),
    )(*parts, res, bsum.reshape(1, od), g.reshape(1, od), b.reshape(1, od))


def _pool_body(hjg_ref, hn_ref, wg_ref, glw_ref, glb_ref, o_ref, *, gb):
    hjg = hjg_ref[...]
    gate = jnp.dot(hjg, wg_ref[...], preferred_element_type=jnp.float32)
    gate = gate - jnp.max(gate, axis=1, keepdims=True)
    ge = jnp.exp(gate)
    w = ge / (jnp.sum(ge, axis=1, keepdims=True) + 1e-16)
    mj = jnp.zeros((gb, DJ), jnp.float32)
    for j in range(J):
        mj = mj + w[:, j:j + 1] * hjg[:, j * DJ:(j + 1) * DJ]
    feat = jnp.concatenate([hn_ref[...], mj], axis=1)
    y = jnp.dot(feat, glw_ref[...], preferred_element_type=jnp.float32)
    o_ref[...] = jnp.maximum(y + glb_ref[...], 0.0)


def _pool_global(hjg, hn, wg_big, gl_W, gl_b, gb=256):
    return pl.pallas_call(
        functools.partial(_pool_body, gb=gb),
        grid=(B // gb,),
        in_specs=[pl.BlockSpec((gb, J * DJ), lambda i: (i, 0)),
                  pl.BlockSpec((gb, 6 * DO), lambda i: (i, 0)),
                  pl.BlockSpec((J * DJ, J), lambda i: (0, 0)),
                  pl.BlockSpec((6 * DO + DJ, GD), lambda i: (0, 0)),
                  pl.BlockSpec((1, GD), lambda i: (0, 0))],
        out_specs=pl.BlockSpec((gb, GD), lambda i: (i, 0)),
        out_shape=jax.ShapeDtypeStruct((B, GD), jnp.float32),
    )(hjg, hn, wg_big, gl_W, gl_b.reshape(1, GD))


def _head_body(f_ref, w1, b1, w2, b2, w3, b3, o_ref):
    h = jnp.dot(f_ref[...], w1[...], preferred_element_type=jnp.float32) + b1[...]
    h = jnp.maximum(h, 0.0)
    h = jnp.dot(h, w2[...], preferred_element_type=jnp.float32) + b2[...]
    h = jnp.maximum(h, 0.0)
    o_ref[...] = jnp.dot(h, w3[...], preferred_element_type=jnp.float32) + b3[...]


def _head(feat, P, bm=2048):
    K = feat.shape[1]
    return pl.pallas_call(
        _head_body,
        grid=(A // bm,),
        in_specs=[pl.BlockSpec((bm, K), lambda i: (i, 0)),
                  pl.BlockSpec((K, 64), lambda i: (0, 0)),
                  pl.BlockSpec((1, 64), lambda i: (0, 0)),
                  pl.BlockSpec((64, 32), lambda i: (0, 0)),
                  pl.BlockSpec((1, 32), lambda i: (0, 0)),
                  pl.BlockSpec((32, 1), lambda i: (0, 0)),
                  pl.BlockSpec((1, 1), lambda i: (0, 0))],
        out_specs=pl.BlockSpec((bm, 1), lambda i: (i, 0)),
        out_shape=jax.ShapeDtypeStruct((A, 1), jnp.float32),
    )(feat, P["q1_W"], P["q1_b"].reshape(1, 64), P["q2_W"],
      P["q2_b"].reshape(1, 32), P["q3_W"], P["q3_b"].reshape(1, 1))


def _fold_attn(W, avec):
    sd, od = W.shape
    C = od // H
    return (W.reshape(sd, H, C) * avec[None]).sum(-1)


def _tabW(p):
    Ws = p["Ws"]
    od = p["b"].shape[0]
    od2 = od // 2
    Wa = jnp.pad(_fold_attn(Ws, p["as"]), ((0, 0), (0, 12)))
    return jnp.concatenate([Ws[:, :od2], Wa, Ws[:, od2:], Wa], axis=1)


def _adW(p):
    return jnp.pad(_fold_attn(p["Wd"], p["ad"]), ((0, 0), (0, 12)))


def kernel(x_job, x_station, x_machine, x_robot, alpha, actions, params, edges):
    P = params
    hj = _mm(x_job, P["lj_W"], P["lj_b"], True)
    hs = _mm(x_station, P["ls_W"], P["ls_b"], True)
    hm = _mm(x_machine, P["lm_W"], P["lm_b"], True)
    hr = _mm(x_robot, P["lr_W"], P["lr_b"], True)

    for jl, ol in (("j1", "o1"), ("j2", "o2")):
        pj = P[jl]
        jrels = ("cl", "ld", "we", "ex", "hd")
        tabs = {}
        for src_x, rels in ((hs, ("cl", "ld")), (hm, ("we", "ex")), (hr, ("hd",))):
            Wall = jnp.concatenate([_tabW(pj[r]) for r in rels], axis=1)
            out = _mm(src_x, Wall, None, False)
            for i, r in enumerate(rels):
                t = out[:, i * 160:(i + 1) * 160]
                tabs[r] = jnp.concatenate([t[:, :80], t[:, 80:]], axis=0)
        Wad_all = jnp.concatenate([_adW(pj[r]) for r in jrels], axis=1)
        adt_all = _mm(hj, Wad_all, None, False)
        parts = []
        for i, r in enumerate(jrels):
            o2, = _sc_gat_multi(
                [(edges[r][0].reshape(-1, 128), edges[r][1].reshape(-1, 128),
                  tabs[r], adt_all[:, i * 16:(i + 1) * 16], NJ)], od=DJ)
            parts += [o2[:NJ], o2[NJ:]]
        bsum = sum(pj[r]["b"] for r in jrels)
        hj = _combine(parts, hj, bsum, pj["ln_g"], pj["ln_b"], od=DJ)

        po = P[ol]
        orels = ("cbl", "li", "nd", "eb", "hb")
        Wall_o = jnp.concatenate([_tabW(po[r]) for r in orels], axis=1)
        outo = _mm(hj, Wall_o, None, False)
        otabs = {}
        for i, r in enumerate(orels):
            t = outo[:, i * 96:(i + 1) * 96]
            otabs[r] = jnp.concatenate([t[:, :48], t[:, 48:]], axis=0)
        adts = {}
        for xd, rels in ((hs, ("cbl", "li")), (hm, ("nd", "eb")), (hr, ("hb",))):
            Wad = jnp.concatenate([_adW(po[r]) for r in rels], axis=1)
            out = _mm(xd, Wad, None, False)
            for i, r in enumerate(rels):
                adts[r] = out[:, i * 16:(i + 1) * 16]
        oparts = {}
        for r, nd in (("cbl", NS), ("li", NS), ("nd", NM), ("eb", NM), ("hb", NR)):
            o2, = _sc_gat_multi(
                [(edges[r][0].reshape(-1, 128), edges[r][1].reshape(-1, 128),
                  otabs[r], adts[r], nd)], od=DO)
            oparts[r] = [o2[:nd], o2[nd:]]
        hs = _combine(oparts["cbl"] + oparts["li"], hs,
                      po["cbl"]["b"] + po["li"]["b"],
                      po["ln_gs"], po["ln_bs"], od=DO)
        hm = _combine(oparts["nd"] + oparts["eb"], hm,
                      po["nd"]["b"] + po["eb"]["b"],
                      po["ln_gm"], po["ln_bm"], od=DO)
        hr = _combine(oparts["hb"], hr, po["hb"]["b"],
                      po["ln_gr"], po["ln_br"], od=DO)

    hjg = hj.reshape(B, J * DJ)
    hn = jnp.concatenate([hs.reshape(B, 3 * DO), hm.reshape(B, 2 * DO), hr], axis=1)
    wg_big = jnp.kron(jnp.eye(J, dtype=jnp.float32), P["gate_W"])
    h_global = _pool_global(hjg, hn, wg_big, P["gl_W"], P["gl_b"])

    job_ids = actions[:, 0]
    graph = job_ids // J
    gji = job_ids + graph * J
    idx = jnp.concatenate([gji, NJ + graph]).astype(jnp.int32).reshape(64, 128)
    tab = jnp.concatenate([hj, h_global], axis=0)
    rowsg = _sc_gather(tab, idx)
    emb = rowsg[:A]
    hg = rowsg[A:]
    feat = jnp.concatenate(
        [emb, hg, actions[:, 1:2].astype(jnp.float32),
         actions[:, 2:3].astype(jnp.float32),
         jnp.broadcast_to(alpha.reshape(1, 1).astype(jnp.float32), (A, 1))],
        axis=1)
    return _head(feat, P)[:, 0]
